# baseline (speedup 1.0000x reference)
import functools

import jax
import jax.numpy as jnp
from jax import lax
from jax.experimental import pallas as pl
from jax.experimental.pallas import tpu as pltpu
from jax.experimental.pallas import tpu_sc as plsc

D = 1024
L = 16
NW = 32
R_CHUNK = 64
BLOCK_ROWS = 1024


def _tc_body(mod_ref, v_ref, l_ref, ov_ref, ol_ref):
    ov_ref[...] = v_ref[...] + mod_ref[0:1, :]
    ol_ref[...] = l_ref[...] + mod_ref[1:2, :]


def _tc_add(v2, l2, mod_emb):
    n = v2.shape[0]
    grid = (n // BLOCK_ROWS,)
    return pl.pallas_call(
        _tc_body,
        grid=grid,
        in_specs=[
            pl.BlockSpec((2, D), lambda i: (0, 0)),
            pl.BlockSpec((BLOCK_ROWS, D), lambda i: (i, 0)),
            pl.BlockSpec((BLOCK_ROWS, D), lambda i: (i, 0)),
        ],
        out_specs=[
            pl.BlockSpec((BLOCK_ROWS, D), lambda i: (i, 0)),
            pl.BlockSpec((BLOCK_ROWS, D), lambda i: (i, 0)),
        ],
        out_shape=[
            jax.ShapeDtypeStruct((n, D), jnp.float32),
            jax.ShapeDtypeStruct((n, D), jnp.float32),
        ],
    )(mod_emb, v2, l2)


def _sc_compute_chunk(buf, m):
    for band in range(D // (16 * L)):
        mjs = [m[pl.ds((band * 16 + jj) * L, L)] for jj in range(16)]

        def row_body(r, _):
            base = r * D + band * 16 * L
            for jj in range(16):
                o = base + jj * L
                buf[pl.ds(o, L)] = buf[pl.ds(o, L)] + mjs[jj]
            return 0

        lax.fori_loop(0, R_CHUNK, row_body, 0)


def _sc_add(v1, l1, mod_emb):
    n_el = v1.shape[0]
    per_w = n_el // NW
    chunks = per_w // (R_CHUNK * D)
    mesh = plsc.VectorSubcoreMesh(core_axis_name="c", subcore_axis_name="s")

    @functools.partial(
        pl.kernel,
        mesh=mesh,
        out_type=[
            jax.ShapeDtypeStruct((n_el,), jnp.float32),
            jax.ShapeDtypeStruct((n_el,), jnp.float32),
        ],
        scratch_types=[
            pltpu.VMEM((R_CHUNK * D,), jnp.float32),
            pltpu.VMEM((D,), jnp.float32),
            pltpu.VMEM((D,), jnp.float32),
        ],
    )
    def k(v_hbm, l_hbm, mod_hbm, ov_hbm, ol_hbm, buf, m0, m1):
        wid = lax.axis_index("s") * 2 + lax.axis_index("c")
        base = wid * per_w
        pltpu.sync_copy(mod_hbm.at[0], m0)
        pltpu.sync_copy(mod_hbm.at[1], m1)

        def chunk_body(c, _):
            off = base + c * (R_CHUNK * D)
            pltpu.sync_copy(v_hbm.at[pl.ds(off, R_CHUNK * D)], buf)
            _sc_compute_chunk(buf, m0)
            pltpu.sync_copy(buf, ov_hbm.at[pl.ds(off, R_CHUNK * D)])
            pltpu.sync_copy(l_hbm.at[pl.ds(off, R_CHUNK * D)], buf)
            _sc_compute_chunk(buf, m1)
            pltpu.sync_copy(buf, ol_hbm.at[pl.ds(off, R_CHUNK * D)])
            return 0

        lax.fori_loop(0, chunks, chunk_body, 0)

    return k(v1, l1, mod_emb)


def kernel(vision, language, mod_emb):
    b, lv, d = vision.shape
    _, lt, _ = language.shape
    v1 = vision.reshape(b * lv * d)
    l1 = language.reshape(b * lt * d)
    ov, ol = _sc_add(v1, l1, mod_emb)
    return ov.reshape(b, lv, d), ol.reshape(b, lt, d)


# speedup vs baseline: 1.1342x; 1.1342x over previous
import functools

import jax
import jax.numpy as jnp
from jax import lax
from jax.experimental import pallas as pl
from jax.experimental.pallas import tpu as pltpu
from jax.experimental.pallas import tpu_sc as plsc

D = 1024
L = 16
NW = 32
R_CHUNK = 16
BLOCK_ROWS = 1024


def _tc_body(mod_ref, v_ref, l_ref, ov_ref, ol_ref):
    ov_ref[...] = v_ref[...] + mod_ref[0:1, :]
    ol_ref[...] = l_ref[...] + mod_ref[1:2, :]


def _tc_add(v2, l2, mod_emb):
    n = v2.shape[0]
    grid = (n // BLOCK_ROWS,)
    return pl.pallas_call(
        _tc_body,
        grid=grid,
        in_specs=[
            pl.BlockSpec((2, D), lambda i: (0, 0)),
            pl.BlockSpec((BLOCK_ROWS, D), lambda i: (i, 0)),
            pl.BlockSpec((BLOCK_ROWS, D), lambda i: (i, 0)),
        ],
        out_specs=[
            pl.BlockSpec((BLOCK_ROWS, D), lambda i: (i, 0)),
            pl.BlockSpec((BLOCK_ROWS, D), lambda i: (i, 0)),
        ],
        out_shape=[
            jax.ShapeDtypeStruct((n, D), jnp.float32),
            jax.ShapeDtypeStruct((n, D), jnp.float32),
        ],
    )(mod_emb, v2, l2)


CH = R_CHUNK * D


def _sc_compute_chunk(src, dst, m):
    for band in range(D // (16 * L)):
        mjs = [m[pl.ds((band * 16 + jj) * L, L)] for jj in range(16)]

        def row_body(r, _):
            base = r * D + band * 16 * L
            for jj in range(16):
                o = base + jj * L
                dst[pl.ds(o, L)] = src[pl.ds(o, L)] + mjs[jj]
            return 0

        lax.fori_loop(0, R_CHUNK, row_body, 0)


def _sc_add(v1, l1, mod_emb):
    n_el = v1.shape[0]
    per_w = n_el // NW
    chunks = per_w // CH
    assert chunks % 2 == 0
    mesh = plsc.VectorSubcoreMesh(core_axis_name="c", subcore_axis_name="s")

    @functools.partial(
        pl.kernel,
        mesh=mesh,
        out_type=[
            jax.ShapeDtypeStruct((n_el,), jnp.float32),
            jax.ShapeDtypeStruct((n_el,), jnp.float32),
        ],
        scratch_types=[
            pltpu.VMEM((CH,), jnp.float32),
            pltpu.VMEM((CH,), jnp.float32),
            pltpu.VMEM((CH,), jnp.float32),
            pltpu.VMEM((CH,), jnp.float32),
            pltpu.VMEM((D,), jnp.float32),
            pltpu.VMEM((D,), jnp.float32),
            pltpu.SemaphoreType.DMA,
            pltpu.SemaphoreType.DMA,
            pltpu.SemaphoreType.DMA,
            pltpu.SemaphoreType.DMA,
        ],
    )
    def k(v_hbm, l_hbm, mod_hbm, ov_hbm, ol_hbm,
          in0, in1, out0, out1, m0, m1, si0, si1, so0, so1):
        wid = lax.axis_index("s") * 2 + lax.axis_index("c")
        base = wid * per_w
        pltpu.sync_copy(mod_hbm.at[0], m0)
        pltpu.sync_copy(mod_hbm.at[1], m1)

        ins, outs, sins, souts = (in0, in1), (out0, out1), (si0, si1), (so0, so1)

        def stream_tensor(src_hbm, dst_hbm, m):
            for b in range(2):
                pltpu.async_copy(src_hbm.at[pl.ds(base + b * CH, CH)],
                                 ins[b], sins[b])

            def body(i, _):
                for b in range(2):
                    c = 2 * i + b
                    off = base + c * CH
                    pltpu.make_async_copy(
                        src_hbm.at[pl.ds(off, CH)], ins[b], sins[b]).wait()
                    @pl.when(i > 0)
                    def _():
                        pltpu.make_async_copy(
                            outs[b], dst_hbm.at[pl.ds(off, CH)], souts[b]).wait()
                    _sc_compute_chunk(ins[b], outs[b], m)
                    @pl.when(c + 2 < chunks)
                    def _():
                        pltpu.async_copy(
                            src_hbm.at[pl.ds(off + 2 * CH, CH)], ins[b], sins[b])
                    pltpu.async_copy(outs[b], dst_hbm.at[pl.ds(off, CH)],
                                     souts[b])
                return 0

            lax.fori_loop(0, chunks // 2, body, 0)
            for b in range(2):
                pltpu.make_async_copy(
                    outs[b], dst_hbm.at[pl.ds(base, CH)], souts[b]).wait()

        stream_tensor(v_hbm, ov_hbm, m0)
        stream_tensor(l_hbm, ol_hbm, m1)

    return k(v1, l1, mod_emb)


def kernel(vision, language, mod_emb):
    b, lv, d = vision.shape
    _, lt, _ = language.shape
    v1 = vision.reshape(b * lv * d)
    l1 = language.reshape(b * lt * d)
    ov, ol = _sc_add(v1, l1, mod_emb)
    return ov.reshape(b, lv, d), ol.reshape(b, lt, d)


# speedup vs baseline: 1.7926x; 1.5806x over previous
import functools

import jax
import jax.numpy as jnp
from jax import lax
from jax.experimental import pallas as pl
from jax.experimental.pallas import tpu as pltpu
from jax.experimental.pallas import tpu_sc as plsc

D = 1024
L = 16
NW = 32
R_CHUNK = 16
BLOCK_ROWS = 1024


def _tc_body(mod_ref, v_ref, l_ref, ov_ref, ol_ref):
    ov_ref[...] = v_ref[...] + mod_ref[0:1, :]
    ol_ref[...] = l_ref[...] + mod_ref[1:2, :]


def _tc_add(v2, l2, mod_emb):
    n = v2.shape[0]
    grid = (n // BLOCK_ROWS,)
    return pl.pallas_call(
        _tc_body,
        grid=grid,
        in_specs=[
            pl.BlockSpec((2, D), lambda i: (0, 0)),
            pl.BlockSpec((BLOCK_ROWS, D), lambda i: (i, 0)),
            pl.BlockSpec((BLOCK_ROWS, D), lambda i: (i, 0)),
        ],
        out_specs=[
            pl.BlockSpec((BLOCK_ROWS, D), lambda i: (i, 0)),
            pl.BlockSpec((BLOCK_ROWS, D), lambda i: (i, 0)),
        ],
        out_shape=[
            jax.ShapeDtypeStruct((n, D), jnp.float32),
            jax.ShapeDtypeStruct((n, D), jnp.float32),
        ],
    )(mod_emb, v2, l2)


CH = R_CHUNK * D


def _sc_compute_chunk(src, dst, m):
    for band in range(D // (16 * L)):
        mjs = [m[pl.ds((band * 16 + jj) * L, L)] for jj in range(16)]

        def row_body(r, _):
            base = r * D + band * 16 * L
            for jj in range(16):
                o = base + jj * L
                dst[pl.ds(o, L)] = src[pl.ds(o, L)] + mjs[jj]
            return 0

        lax.fori_loop(0, R_CHUNK, row_body, 0)


def _sc_add_one(x1, mod_emb, row):
    n_el = x1.shape[0]
    per_w = n_el // NW
    chunks = per_w // CH
    assert chunks % 2 == 0
    mesh = plsc.VectorSubcoreMesh(core_axis_name="c", subcore_axis_name="s")

    @functools.partial(
        pl.kernel,
        mesh=mesh,
        out_type=jax.ShapeDtypeStruct((n_el,), jnp.float32),
        scratch_types=[
            pltpu.VMEM((CH,), jnp.float32),
            pltpu.VMEM((CH,), jnp.float32),
            pltpu.VMEM((CH,), jnp.float32),
            pltpu.VMEM((CH,), jnp.float32),
            pltpu.VMEM((D,), jnp.float32),
            pltpu.SemaphoreType.DMA,
            pltpu.SemaphoreType.DMA,
            pltpu.SemaphoreType.DMA,
            pltpu.SemaphoreType.DMA,
        ],
    )
    def k(x_hbm, mod_hbm, out_hbm,
          in0, in1, out0, out1, m0, si0, si1, so0, so1):
        wid = lax.axis_index("s") * 2 + lax.axis_index("c")
        base = wid * per_w
        pltpu.sync_copy(mod_hbm.at[row], m0)

        ins, outs, sins, souts = (in0, in1), (out0, out1), (si0, si1), (so0, so1)

        for b in range(2):
            pltpu.async_copy(x_hbm.at[pl.ds(base + b * CH, CH)],
                             ins[b], sins[b])

        def body(i, _):
            for b in range(2):
                c = 2 * i + b
                off = base + c * CH
                pltpu.make_async_copy(
                    x_hbm.at[pl.ds(off, CH)], ins[b], sins[b]).wait()
                @pl.when(i > 0)
                def _():
                    pltpu.make_async_copy(
                        outs[b], out_hbm.at[pl.ds(off, CH)], souts[b]).wait()
                _sc_compute_chunk(ins[b], outs[b], m0)
                @pl.when(c + 2 < chunks)
                def _():
                    pltpu.async_copy(
                        x_hbm.at[pl.ds(off + 2 * CH, CH)], ins[b], sins[b])
                pltpu.async_copy(outs[b], out_hbm.at[pl.ds(off, CH)],
                                 souts[b])
            return 0

        lax.fori_loop(0, chunks // 2, body, 0)
        for b in range(2):
            pltpu.make_async_copy(
                outs[b], out_hbm.at[pl.ds(base, CH)], souts[b]).wait()

    return k(x1, mod_emb)


def _tc_body_one(mod_ref, x_ref, o_ref):
    o_ref[...] = x_ref[...] + mod_ref[1:2, :]


def _tc_add_one(x2, mod_emb):
    n = x2.shape[0]
    return pl.pallas_call(
        _tc_body_one,
        grid=(n // BLOCK_ROWS,),
        in_specs=[
            pl.BlockSpec((2, D), lambda i: (0, 0)),
            pl.BlockSpec((BLOCK_ROWS, D), lambda i: (i, 0)),
        ],
        out_specs=pl.BlockSpec((BLOCK_ROWS, D), lambda i: (i, 0)),
        out_shape=jax.ShapeDtypeStruct((n, D), jnp.float32),
    )(mod_emb, x2)


def kernel(vision, language, mod_emb):
    b, lv, d = vision.shape
    _, lt, _ = language.shape
    ov = _sc_add_one(vision.reshape(b * lv * d), mod_emb, 0)
    ol = _tc_add_one(language.reshape(b * lt, d), mod_emb)
    return ov.reshape(b, lv, d), ol.reshape(b, lt, d)


# speedup vs baseline: 3.8890x; 2.1694x over previous
import functools

import jax
import jax.numpy as jnp
from jax import lax
from jax.experimental import pallas as pl
from jax.experimental.pallas import tpu as pltpu
from jax.experimental.pallas import tpu_sc as plsc

D = 1024
L = 16
NW = 32
R_CHUNK = 16
BLOCK_ROWS = 1024


def _tc_body_one(mod_ref, x_ref, o_ref):
    o_ref[...] = x_ref[...] + mod_ref[1:2, :]


def _tc_add_one(x2, mod_emb):
    n = x2.shape[0]
    return pl.pallas_call(
        _tc_body_one,
        grid=(n // BLOCK_ROWS,),
        in_specs=[
            pl.BlockSpec((2, D), lambda i: (0, 0)),
            pl.BlockSpec((BLOCK_ROWS, D), lambda i: (i, 0)),
        ],
        out_specs=pl.BlockSpec((BLOCK_ROWS, D), lambda i: (i, 0)),
        out_shape=jax.ShapeDtypeStruct((n, D), jnp.float32),
    )(mod_emb, x2)


def _sc_compute_chunk(src, dst, m):
    for band in range(D // (16 * L)):
        mjs = [m[pl.ds((band * 16 + jj) * L, L)] for jj in range(16)]

        def row_body(r, _):
            base = band * 16 * L
            for jj in range(16):
                o = base + jj * L
                dst[r, pl.ds(o, L)] = src[r, pl.ds(o, L)] + mjs[jj]
            return 0

        lax.fori_loop(0, R_CHUNK, row_body, 0)


def _sc_add_one(x2, mod_emb, row):
    n = x2.shape[0]
    rows_w = n // NW
    chunks = rows_w // R_CHUNK
    assert chunks % 2 == 0
    mesh = plsc.VectorSubcoreMesh(core_axis_name="c", subcore_axis_name="s")

    @functools.partial(
        pl.kernel,
        mesh=mesh,
        out_type=jax.ShapeDtypeStruct((n, D), jnp.float32),
        scratch_types=[
            pltpu.VMEM((R_CHUNK, D), jnp.float32),
            pltpu.VMEM((R_CHUNK, D), jnp.float32),
            pltpu.VMEM((R_CHUNK, D), jnp.float32),
            pltpu.VMEM((R_CHUNK, D), jnp.float32),
            pltpu.VMEM((D,), jnp.float32),
            pltpu.SemaphoreType.DMA,
            pltpu.SemaphoreType.DMA,
            pltpu.SemaphoreType.DMA,
            pltpu.SemaphoreType.DMA,
        ],
    )
    def k(x_hbm, mod_hbm, out_hbm,
          in0, in1, out0, out1, m0, si0, si1, so0, so1):
        wid = lax.axis_index("s") * 2 + lax.axis_index("c")
        base = wid * rows_w
        pltpu.sync_copy(mod_hbm.at[row], m0)

        ins, outs, sins, souts = (in0, in1), (out0, out1), (si0, si1), (so0, so1)

        for b in range(2):
            pltpu.async_copy(x_hbm.at[pl.ds(base + b * R_CHUNK, R_CHUNK)],
                             ins[b], sins[b])

        def body(i, _):
            for b in range(2):
                c = 2 * i + b
                off = base + c * R_CHUNK
                pltpu.make_async_copy(
                    x_hbm.at[pl.ds(off, R_CHUNK)], ins[b], sins[b]).wait()
                @pl.when(i > 0)
                def _():
                    pltpu.make_async_copy(
                        outs[b], out_hbm.at[pl.ds(off, R_CHUNK)],
                        souts[b]).wait()
                _sc_compute_chunk(ins[b], outs[b], m0)
                @pl.when(c + 2 < chunks)
                def _():
                    pltpu.async_copy(
                        x_hbm.at[pl.ds(off + 2 * R_CHUNK, R_CHUNK)],
                        ins[b], sins[b])
                pltpu.async_copy(outs[b], out_hbm.at[pl.ds(off, R_CHUNK)],
                                 souts[b])
            return 0

        lax.fori_loop(0, chunks // 2, body, 0)
        for b in range(2):
            pltpu.make_async_copy(
                outs[b], out_hbm.at[pl.ds(base, R_CHUNK)], souts[b]).wait()

    return k(x2, mod_emb)


def kernel(vision, language, mod_emb):
    b, lv, d = vision.shape
    _, lt, _ = language.shape
    ov = _sc_add_one(vision.reshape(b * lv, d), mod_emb, 0)
    ol = _tc_add_one(language.reshape(b * lt, d), mod_emb)
    return ov.reshape(b, lv, d), ol.reshape(b, lt, d)


# speedup vs baseline: 3.9336x; 1.0115x over previous
import functools

import jax
import jax.numpy as jnp
from jax import lax
from jax.experimental import pallas as pl
from jax.experimental.pallas import tpu as pltpu
from jax.experimental.pallas import tpu_sc as plsc

D = 1024
L = 16
NW = 32
R_CHUNK = 8
NBUF = 4
BLOCK_ROWS = 1024


def _tc_body_one(mod_ref, x_ref, o_ref):
    o_ref[...] = x_ref[...] + mod_ref[1:2, :]


def _tc_add_one(x2, mod_emb):
    n = x2.shape[0]
    return pl.pallas_call(
        _tc_body_one,
        grid=(n // BLOCK_ROWS,),
        in_specs=[
            pl.BlockSpec((2, D), lambda i: (0, 0)),
            pl.BlockSpec((BLOCK_ROWS, D), lambda i: (i, 0)),
        ],
        out_specs=pl.BlockSpec((BLOCK_ROWS, D), lambda i: (i, 0)),
        out_shape=jax.ShapeDtypeStruct((n, D), jnp.float32),
    )(mod_emb, x2)


def _sc_compute_chunk(src, dst, m):
    for band in range(D // (16 * L)):
        mjs = [m[pl.ds((band * 16 + jj) * L, L)] for jj in range(16)]

        def row_body(r, _):
            base = band * 16 * L
            for jj in range(16):
                o = base + jj * L
                dst[r, pl.ds(o, L)] = src[r, pl.ds(o, L)] + mjs[jj]
            return 0

        lax.fori_loop(0, R_CHUNK, row_body, 0)


def _sc_add_one(x2, mod_emb, row):
    n = x2.shape[0]
    rows_w = n // NW
    chunks = rows_w // R_CHUNK
    assert chunks % NBUF == 0
    mesh = plsc.VectorSubcoreMesh(core_axis_name="c", subcore_axis_name="s")

    @functools.partial(
        pl.kernel,
        mesh=mesh,
        out_type=jax.ShapeDtypeStruct((n, D), jnp.float32),
        scratch_types=(
            [pltpu.VMEM((R_CHUNK, D), jnp.float32)] * (2 * NBUF)
            + [pltpu.VMEM((D,), jnp.float32)]
            + [pltpu.SemaphoreType.DMA] * (2 * NBUF)
        ),
    )
    def k(x_hbm, mod_hbm, out_hbm, *scratch):
        ins = scratch[:NBUF]
        outs = scratch[NBUF:2 * NBUF]
        m0 = scratch[2 * NBUF]
        sins = scratch[2 * NBUF + 1:2 * NBUF + 1 + NBUF]
        souts = scratch[2 * NBUF + 1 + NBUF:]
        wid = lax.axis_index("s") * 2 + lax.axis_index("c")
        base = wid * rows_w
        pltpu.sync_copy(mod_hbm.at[row], m0)

        for b in range(NBUF):
            pltpu.async_copy(x_hbm.at[pl.ds(base + b * R_CHUNK, R_CHUNK)],
                             ins[b], sins[b])

        def body(i, _):
            for b in range(NBUF):
                c = NBUF * i + b
                off = base + c * R_CHUNK
                pltpu.make_async_copy(
                    x_hbm.at[pl.ds(off, R_CHUNK)], ins[b], sins[b]).wait()
                @pl.when(i > 0)
                def _():
                    pltpu.make_async_copy(
                        outs[b], out_hbm.at[pl.ds(off, R_CHUNK)],
                        souts[b]).wait()
                _sc_compute_chunk(ins[b], outs[b], m0)
                @pl.when(c + NBUF < chunks)
                def _():
                    pltpu.async_copy(
                        x_hbm.at[pl.ds(off + NBUF * R_CHUNK, R_CHUNK)],
                        ins[b], sins[b])
                pltpu.async_copy(outs[b], out_hbm.at[pl.ds(off, R_CHUNK)],
                                 souts[b])
            return 0

        lax.fori_loop(0, chunks // NBUF, body, 0)
        for b in range(NBUF):
            pltpu.make_async_copy(
                outs[b], out_hbm.at[pl.ds(base, R_CHUNK)], souts[b]).wait()

    return k(x2, mod_emb)


def kernel(vision, language, mod_emb):
    b, lv, d = vision.shape
    _, lt, _ = language.shape
    ov = _sc_add_one(vision.reshape(b * lv, d), mod_emb, 0)
    ol = _tc_add_one(language.reshape(b * lt, d), mod_emb)
    return ov.reshape(b, lv, d), ol.reshape(b, lt, d)
